# baseline (speedup 1.0000x reference)
import jax
import jax.numpy as jnp
from jax import lax
from jax.experimental import pallas as pl
from jax.experimental.pallas import tpu as pltpu
from jax.experimental.pallas import tpu_sc as plsc

_NC, _NS = 2, 16
_NW = _NC * _NS
_ROWS_PER_W = 1
_LQ = 5440
_S = 5440
_SPAD = 5504
_KPC = 2
_NCHUNK = 16 // _KPC
_NQV = _LQ // 16

_WIDTHS = (64, 32, 16, 8)
_BASES = (0, 4096, 5120, 5376)


def _sc_body(loc_hbm, aw_hbm, out_hbm,
             locb0, awb0, locb1, awb1, acc, sem0, sem1):
    wid = lax.axis_index("s") * _NC + lax.axis_index("c")

    locbufs = (locb0, locb1)
    awbufs = (awb0, awb1)
    sems = (sem0, sem1)

    def copy_chunk(r, c, slot):
        d0 = pltpu.async_copy(
            loc_hbm.at[r, pl.ds(c * _KPC, _KPC)], locbufs[slot], sems[slot])
        d1 = pltpu.async_copy(
            aw_hbm.at[r, pl.ds(c * _KPC, _KPC)], awbufs[slot], sems[slot])
        return d0, d1

    for j in range(_ROWS_PER_W):
        r = wid * _ROWS_PER_W + j
        pending = copy_chunk(r, 0, 0)

        def _zero(i, carry):
            acc[pl.ds(i * 16, 16)] = jnp.zeros((16,), jnp.float32)
            return carry
        lax.fori_loop(0, _SPAD // 16, _zero, 0)

        for c in range(_NCHUNK):
            cur = c % 2
            if c + 1 < _NCHUNK:
                nxt_pending = copy_chunk(r, c + 1, 1 - cur)
            pending[0].wait()
            pending[1].wait()
            locb = locbufs[cur]
            awb = awbufs[cur]

            for kp in range(_KPC):
                lev = (c * _KPC + kp) // 4
                w = _WIDTHS[lev]
                base = _BASES[lev]
                wf = float(w)

                @plsc.parallel_loop(0, _NQV, unroll=5)
                def _qv(i, kp=kp, w=w, base=base, wf=wf):
                    qs = pl.ds(i * 16, 16)
                    x = locb[kp, 0, qs]
                    y = locb[kp, 1, qs]
                    aw = awb[kp, qs]
                    xs = x * wf
                    ys = y * wf
                    cx = xs.astype(jnp.int32)
                    cy = ys.astype(jnp.int32)
                    fx = xs - cx.astype(jnp.float32)
                    fy = ys - cy.astype(jnp.float32)
                    gx = 1.0 - fx
                    gy = 1.0 - fy
                    wl = aw * gx
                    wh = aw * fx
                    i0 = cy * w + cx + base
                    iw = i0 + w
                    mx = cx < (w - 1)
                    my = cy < (w - 1)
                    plsc.addupdate_scatter(acc, [i0], wl * gy)
                    plsc.addupdate_scatter(acc, [iw], wl * fy, mask=my)
                    plsc.addupdate_scatter(acc, [i0 + 1], wh * gy, mask=mx)
                    plsc.addupdate_scatter(acc, [iw + 1], wh * fy,
                                           mask=mx & my)

            if c + 1 < _NCHUNK:
                pending = nxt_pending

        pltpu.sync_copy(acc.at[pl.ds(0, _S)], out_hbm.at[r])


def _run(loc, aw):
    mesh = plsc.VectorSubcoreMesh(core_axis_name="c", subcore_axis_name="s",
                                  num_cores=_NC, num_subcores=_NS)
    f = pl.kernel(
        _sc_body,
        out_type=jax.ShapeDtypeStruct((_NW * _ROWS_PER_W, _S), jnp.float32),
        mesh=mesh,
        scratch_types=[
            pltpu.VMEM((_KPC, 2, _LQ), jnp.float32),
            pltpu.VMEM((_KPC, _LQ), jnp.float32),
            pltpu.VMEM((_KPC, 2, _LQ), jnp.float32),
            pltpu.VMEM((_KPC, _LQ), jnp.float32),
            pltpu.VMEM((_SPAD,), jnp.float32),
            pltpu.SemaphoreType.DMA,
            pltpu.SemaphoreType.DMA,
        ],
        compiler_params=pltpu.CompilerParams(needs_layout_passes=False,
                                             use_tc_tiling_on_sc=False),
    )
    return f(loc, aw)


def kernel(spatial_shapes, level_start_index, sampling_locations,
           attention_weights):
    n, nl, lq, nh, nlev, npt, _ = sampling_locations.shape
    outs = []
    for g in range(3):
        sl = sampling_locations[:, 2 * g:2 * g + 2]
        awg = attention_weights[:, 2 * g:2 * g + 2]
        loc = (jnp.transpose(sl, (0, 1, 3, 4, 5, 6, 2))
               .reshape(n * 2 * nh, nlev * npt, 2, lq))
        aw = (jnp.transpose(awg, (0, 1, 3, 4, 5, 2))
              .reshape(n * 2 * nh, nlev * npt, lq))
        outs.append(_run(loc, aw))
    out = jnp.stack(outs, axis=0).reshape(3, n, 2, nh, _S)
    out = jnp.transpose(out, (1, 0, 2, 3, 4)).reshape(n, nl, nh, _S)
    return out


# speedup vs baseline: 1.0028x; 1.0028x over previous
import jax
import jax.numpy as jnp
from jax import lax
from jax.experimental import pallas as pl
from jax.experimental.pallas import tpu as pltpu
from jax.experimental.pallas import tpu_sc as plsc

_NC, _NS = 2, 16
_NW = _NC * _NS
_ROWS_PER_W = 1
_LQ = 5440
_S = 5440
_SPAD = 5504
_KPC = 2
_NCHUNK = 16 // _KPC
_NQV = _LQ // 16

_WIDTHS = (64, 32, 16, 8)
_BASES = (0, 4096, 5120, 5376)


def _sc_body(loc_hbm, aw_hbm, out_hbm,
             locb0, awb0, locb1, awb1, acc, sem0, sem1):
    wid = lax.axis_index("s") * _NC + lax.axis_index("c")

    locbufs = (locb0, locb1)
    awbufs = (awb0, awb1)
    sems = (sem0, sem1)

    def copy_chunk(r, c, slot):
        d0 = pltpu.async_copy(
            loc_hbm.at[r, pl.ds(c * _KPC, _KPC)], locbufs[slot], sems[slot])
        d1 = pltpu.async_copy(
            aw_hbm.at[r, pl.ds(c * _KPC, _KPC)], awbufs[slot], sems[slot])
        return d0, d1

    for j in range(_ROWS_PER_W):
        r = wid * _ROWS_PER_W + j
        pending = copy_chunk(r, 0, 0)

        def _zero(i, carry):
            acc[pl.ds(i * 16, 16)] = jnp.zeros((16,), jnp.float32)
            return carry
        lax.fori_loop(0, _SPAD // 16, _zero, 0)

        for c in range(_NCHUNK):
            cur = c % 2
            if c + 1 < _NCHUNK:
                nxt_pending = copy_chunk(r, c + 1, 1 - cur)
            pending[0].wait()
            pending[1].wait()
            locb = locbufs[cur]
            awb = awbufs[cur]

            for kp in range(_KPC):
                lev = (c * _KPC + kp) // 4
                w = _WIDTHS[lev]
                base = _BASES[lev]
                wf = float(w)

                @plsc.parallel_loop(0, _NQV, unroll=4)
                def _qv(i, kp=kp, w=w, base=base, wf=wf):
                    qs = pl.ds(i * 16, 16)
                    x = locb[kp, 0, qs]
                    y = locb[kp, 1, qs]
                    aw = awb[kp, qs]
                    xs = x * wf
                    ys = y * wf
                    cx = xs.astype(jnp.int32)
                    cy = ys.astype(jnp.int32)
                    fx = xs - cx.astype(jnp.float32)
                    fy = ys - cy.astype(jnp.float32)
                    gx = 1.0 - fx
                    gy = 1.0 - fy
                    wl = aw * gx
                    wh = aw * fx
                    i0 = cy * w + cx + base
                    iw = i0 + w
                    mx = cx < (w - 1)
                    my = cy < (w - 1)
                    plsc.addupdate_scatter(acc, [i0], wl * gy)
                    plsc.addupdate_scatter(acc, [iw], wl * fy, mask=my)
                    plsc.addupdate_scatter(acc, [i0 + 1], wh * gy, mask=mx)
                    plsc.addupdate_scatter(acc, [iw + 1], wh * fy,
                                           mask=mx & my)

            if c + 1 < _NCHUNK:
                pending = nxt_pending

        pltpu.sync_copy(acc.at[pl.ds(0, _S)], out_hbm.at[r])


def _run(loc, aw):
    mesh = plsc.VectorSubcoreMesh(core_axis_name="c", subcore_axis_name="s",
                                  num_cores=_NC, num_subcores=_NS)
    f = pl.kernel(
        _sc_body,
        out_type=jax.ShapeDtypeStruct((_NW * _ROWS_PER_W, _S), jnp.float32),
        mesh=mesh,
        scratch_types=[
            pltpu.VMEM((_KPC, 2, _LQ), jnp.float32),
            pltpu.VMEM((_KPC, _LQ), jnp.float32),
            pltpu.VMEM((_KPC, 2, _LQ), jnp.float32),
            pltpu.VMEM((_KPC, _LQ), jnp.float32),
            pltpu.VMEM((_SPAD,), jnp.float32),
            pltpu.SemaphoreType.DMA,
            pltpu.SemaphoreType.DMA,
        ],
        compiler_params=pltpu.CompilerParams(needs_layout_passes=False,
                                             use_tc_tiling_on_sc=False),
    )
    return f(loc, aw)


def kernel(spatial_shapes, level_start_index, sampling_locations,
           attention_weights):
    n, nl, lq, nh, nlev, npt, _ = sampling_locations.shape
    outs = []
    for g in range(3):
        sl = sampling_locations[:, 2 * g:2 * g + 2]
        awg = attention_weights[:, 2 * g:2 * g + 2]
        loc = (jnp.transpose(sl, (0, 1, 3, 4, 5, 6, 2))
               .reshape(n * 2 * nh, nlev * npt, 2, lq))
        aw = (jnp.transpose(awg, (0, 1, 3, 4, 5, 2))
              .reshape(n * 2 * nh, nlev * npt, lq))
        outs.append(_run(loc, aw).reshape(n, 2, nh, _S))
    return jnp.concatenate(outs, axis=1)
